# baseline (speedup 1.0000x reference)
import jax
import jax.numpy as jnp
from jax.experimental import pallas as pl
from jax.experimental.pallas import tpu as pltpu

_GROUP = 400
_SUB = 10
_CHUNK = _GROUP // _SUB
_NSLOT = 3


def _gcn_kernel(x_ref, adj_ref, w_ref, b_ref, out_ref, bufs, support, obuf, sems, osems):
    n = x_ref.shape[0]
    d_out = w_ref.shape[1]
    ngroup = n // _GROUP

    def sub_copy(g, s, slot):
        return pltpu.make_async_copy(
            adj_ref.at[pl.ds(g * _GROUP + s * _CHUNK, _CHUNK), :],
            bufs.at[slot, pl.ds(s * _CHUNK, _CHUNK), :],
            sems.at[slot],
        )

    def out_copy(g, oslot):
        return pltpu.make_async_copy(
            obuf.at[oslot],
            out_ref.at[pl.ds(g * _GROUP, _GROUP), :],
            osems.at[oslot],
        )

    for g in range(_NSLOT - 1):
        for s in range(_SUB):
            sub_copy(g, s, g).start()

    support[...] = jnp.dot(
        x_ref[...], w_ref[...], preferred_element_type=jnp.float32
    )

    def body(g, carry):
        slot = jax.lax.rem(g, _NSLOT)
        nxt = g + _NSLOT - 1

        @pl.when(nxt < ngroup)
        def _():
            nslot = jax.lax.rem(nxt, _NSLOT)
            for s in range(_SUB):
                sub_copy(nxt, s, nslot).start()

        for s in range(_SUB):
            sub_copy(g, s, slot).wait()

        oslot = jax.lax.rem(g, 2)

        @pl.when(g >= 2)
        def _():
            out_copy(g - 2, oslot).wait()

        obuf[oslot] = (
            jnp.dot(bufs[slot], support[...], preferred_element_type=jnp.float32)
            + b_ref[...]
        )
        out_copy(g, oslot).start()
        return carry

    jax.lax.fori_loop(0, ngroup, body, 0)
    out_copy(ngroup - 2, jax.lax.rem(ngroup - 2, 2)).wait()
    out_copy(ngroup - 1, jax.lax.rem(ngroup - 1, 2)).wait()


def kernel(input, adj, W, b):
    n, d_in = input.shape
    d_out = W.shape[1]
    b2 = b.reshape(1, d_out)
    return pl.pallas_call(
        _gcn_kernel,
        in_specs=[
            pl.BlockSpec(memory_space=pltpu.MemorySpace.VMEM),
            pl.BlockSpec(memory_space=pltpu.MemorySpace.HBM),
            pl.BlockSpec(memory_space=pltpu.MemorySpace.VMEM),
            pl.BlockSpec(memory_space=pltpu.MemorySpace.VMEM),
        ],
        out_specs=pl.BlockSpec(memory_space=pltpu.MemorySpace.HBM),
        out_shape=jax.ShapeDtypeStruct((n, d_out), jnp.float32),
        scratch_shapes=[
            pltpu.VMEM((_NSLOT, _GROUP, n), jnp.float32),
            pltpu.VMEM((n, d_out), jnp.float32),
            pltpu.VMEM((2, _GROUP, d_out), jnp.float32),
            pltpu.SemaphoreType.DMA((_NSLOT,)),
            pltpu.SemaphoreType.DMA((2,)),
        ],
    )(input, adj, W, b2)


# speedup vs baseline: 1.0338x; 1.0338x over previous
import jax
import jax.numpy as jnp
from jax.experimental import pallas as pl
from jax.experimental.pallas import tpu as pltpu

_BM = 400


def _gcn_block(x_ref, adj_ref, w_ref, b_ref, out_ref):
    t = jnp.dot(adj_ref[...], x_ref[...], preferred_element_type=jnp.float32)
    out_ref[...] = (
        jnp.dot(t, w_ref[...], preferred_element_type=jnp.float32) + b_ref[...]
    )


def kernel(input, adj, W, b):
    n, d_in = input.shape
    d_out = W.shape[1]
    bm = _BM
    b2 = b.reshape(1, d_out)
    return pl.pallas_call(
        _gcn_block,
        grid=(n // bm,),
        in_specs=[
            pl.BlockSpec((n, d_in), lambda m: (0, 0)),
            pl.BlockSpec((bm, n), lambda m: (m, 0)),
            pl.BlockSpec((d_in, d_out), lambda m: (0, 0)),
            pl.BlockSpec((1, d_out), lambda m: (0, 0)),
        ],
        out_specs=pl.BlockSpec((bm, d_out), lambda m: (m, 0)),
        out_shape=jax.ShapeDtypeStruct((n, d_out), jnp.float32),
        compiler_params=pltpu.CompilerParams(
            dimension_semantics=("parallel",),
        ),
    )(input, adj, W, b2)
